# baseline (speedup 1.0000x reference)
import functools

import jax
import jax.numpy as jnp
from jax import lax
from jax.experimental import pallas as pl
from jax.experimental.pallas import tpu as pltpu
from jax.experimental.pallas import tpu_sc as plsc

NC = 2
NS = 16
NW = NC * NS
CHUNK = 2048
DIM = 16
LANES = 16
TILE_R = 8
TILE_C = 128
PLANE = CHUNK * TILE_R


@functools.lru_cache(maxsize=None)
def _make_gather(n_atom_cols, n_edge_cols, n_feat_atom, n_feat_edge,
                 atom_rows, bond_rows):
    a_tc = n_atom_cols // TILE_C
    e_tc = n_edge_cols // TILE_C
    a_nch = -(-n_atom_cols // CHUNK)
    e_nch = -(-n_edge_cols // CHUNK)
    a_tasks = n_feat_atom * a_nch
    e_tasks = n_feat_edge * e_nch

    mesh = plsc.VectorSubcoreMesh(core_axis_name="c", subcore_axis_name="s")

    @functools.partial(
        pl.kernel,
        out_type=(
            jax.ShapeDtypeStruct((n_feat_atom * DIM * n_atom_cols,), jnp.float32),
            jax.ShapeDtypeStruct((n_feat_edge * DIM * n_edge_cols,), jnp.float32),
        ),
        mesh=mesh,
        scratch_types=[
            pltpu.VMEM((atom_rows * DIM,), jnp.float32),
            pltpu.VMEM((bond_rows * DIM,), jnp.float32),
            pltpu.VMEM((2, CHUNK), jnp.int32),
            pltpu.VMEM((2, 2, PLANE), jnp.float32),
            pltpu.SemaphoreType.DMA,
            pltpu.SemaphoreType.DMA,
            pltpu.SemaphoreType.DMA,
            pltpu.SemaphoreType.DMA,
        ],
        compiler_params=pltpu.CompilerParams(
            use_tc_tiling_on_sc=False, needs_layout_passes=False),
    )
    def gather_kernel(atab, xidx, btab, eidx, xout, eout,
                      atab_v, btab_v, idx_v, rows_v,
                      sem_i0, sem_i1, sem_o0, sem_o1):
        wid = lax.axis_index("s") * NC + lax.axis_index("c")
        pltpu.sync_copy(atab, atab_v)
        pltpu.sync_copy(btab, btab_v)
        sem_i = (sem_i0, sem_i1)
        sem_o = (sem_o0, sem_o1)

        def run(tab_v, idxs, out, n_tasks, nch, ncols, ntc):
            n_max = -(-n_tasks // NW)
            n_pair = -(-n_max // 2)
            total = 2 * n_pair

            def chunk_of(k):
                t = jnp.minimum(wid + k * NW, n_tasks - 1)
                f = t // nch
                n0 = jnp.minimum((t % nch) * CHUNK, ncols - CHUNK)
                return f, n0

            def start_idx(k, s):
                f, n0 = chunk_of(k)
                pltpu.async_copy(idxs.at[pl.ds(f * ncols + n0, CHUNK)],
                                 idx_v.at[s], sem_i[s])

            def wait_idx(s):
                pltpu.make_async_copy(idxs.at[pl.ds(0, CHUNK)],
                                      idx_v.at[s], sem_i[s]).wait()

            def wait_out(s):
                for tr in range(2):
                    pltpu.make_async_copy(rows_v.at[s, tr],
                                          out.at[pl.ds(0, PLANE)],
                                          sem_o[s]).wait()

            start_idx(0, 0)
            start_idx(1, 1)

            def pair_body(kk, carry):
                for s in range(2):
                    k = kk * 2 + s
                    wait_idx(s)

                    @pl.when(k >= 2)
                    def _():
                        wait_out(s)

                    @plsc.parallel_loop(0, CHUNK // LANES, unroll=2)
                    def row_body(j):
                        iv = idx_v[s, pl.ds(j * LANES, LANES)]
                        base = iv * DIM
                        off = (j // 8) * (TILE_R * TILE_C) + (j % 8) * LANES
                        skew = 4
                        vals = []

                        def store(d):
                            rows_v[s, d // TILE_R,
                                   pl.ds(off + (d % TILE_R) * TILE_C,
                                         LANES)] = vals[d]

                        for d in range(DIM):
                            vals.append(plsc.load_gather(tab_v, [base + d]))
                            if d >= skew:
                                store(d - skew)
                        for d in range(DIM - skew, DIM):
                            store(d)

                    f, n0 = chunk_of(k)
                    for tr in range(2):
                        q0 = ((f * 2 + tr) * ntc + n0 // TILE_C) * (TILE_R * TILE_C)
                        pltpu.async_copy(rows_v.at[s, tr],
                                         out.at[pl.ds(q0, PLANE)], sem_o[s])

                    @pl.when(k + 2 < total)
                    def _():
                        start_idx(k + 2, s)
                return carry

            lax.fori_loop(0, n_pair, pair_body, 0)
            for s in range(2):
                wait_out(s)

        run(atab_v, xidx, xout, a_tasks, a_nch, n_atom_cols, a_tc)
        run(btab_v, eidx, eout, e_tasks, e_nch, n_edge_cols, e_tc)

    return gather_kernel


def kernel(x, edge_attr, atom_table, bond_table):
    n_atom, f_atom = x.shape
    n_edge, f_edge = edge_attr.shape
    a_cols = -(-n_atom // TILE_C) * TILE_C
    e_cols = -(-n_edge // TILE_C) * TILE_C

    xt = jnp.pad(x.T.astype(jnp.int32), ((0, 0), (0, a_cols - n_atom)))
    et = jnp.pad(edge_attr.T.astype(jnp.int32), ((0, 0), (0, e_cols - n_edge)))

    gk = _make_gather(a_cols, e_cols, f_atom, f_edge,
                      atom_table.shape[0], bond_table.shape[0])
    xo, eo = gk(atom_table.reshape(-1), xt.reshape(-1),
                bond_table.reshape(-1), et.reshape(-1))

    x_emb = (xo.reshape(f_atom, 2, a_cols // TILE_C, TILE_R, TILE_C)
             .transpose(2, 4, 0, 1, 3)
             .reshape(a_cols, f_atom, DIM)[:n_atom])
    e_emb = (eo.reshape(f_edge, 2, e_cols // TILE_C, TILE_R, TILE_C)
             .transpose(2, 4, 0, 1, 3)
             .reshape(e_cols, f_edge, DIM)[:n_edge])
    return (x_emb, e_emb)


# speedup vs baseline: 1.0625x; 1.0625x over previous
import functools

import jax
import jax.numpy as jnp
from jax import lax
from jax.experimental import pallas as pl
from jax.experimental.pallas import tpu as pltpu
from jax.experimental.pallas import tpu_sc as plsc

NC = 2
NS = 16
NW = NC * NS
CHUNK = 2560
DIM = 16
LANES = 16
TILE_R = 8
TILE_C = 128
PLANE = CHUNK * TILE_R


@functools.lru_cache(maxsize=None)
def _make_gather(n_atom_cols, n_edge_cols, n_feat_atom, n_feat_edge,
                 atom_rows, bond_rows):
    a_tc = n_atom_cols // TILE_C
    e_tc = n_edge_cols // TILE_C
    a_nch = -(-n_atom_cols // CHUNK)
    e_nch = -(-n_edge_cols // CHUNK)
    a_tasks = n_feat_atom * a_nch
    e_tasks = n_feat_edge * e_nch

    mesh = plsc.VectorSubcoreMesh(core_axis_name="c", subcore_axis_name="s")

    @functools.partial(
        pl.kernel,
        out_type=(
            jax.ShapeDtypeStruct((n_feat_atom * DIM * n_atom_cols,), jnp.float32),
            jax.ShapeDtypeStruct((n_feat_edge * DIM * n_edge_cols,), jnp.float32),
        ),
        mesh=mesh,
        scratch_types=[
            pltpu.VMEM((atom_rows * DIM,), jnp.float32),
            pltpu.VMEM((bond_rows * DIM,), jnp.float32),
            pltpu.VMEM((2, CHUNK), jnp.int32),
            pltpu.VMEM((2, 2, PLANE), jnp.float32),
            pltpu.SemaphoreType.DMA,
            pltpu.SemaphoreType.DMA,
            pltpu.SemaphoreType.DMA,
            pltpu.SemaphoreType.DMA,
        ],
        compiler_params=pltpu.CompilerParams(
            use_tc_tiling_on_sc=False, needs_layout_passes=False),
    )
    def gather_kernel(atab, xidx, btab, eidx, xout, eout,
                      atab_v, btab_v, idx_v, rows_v,
                      sem_i0, sem_i1, sem_o0, sem_o1):
        wid = lax.axis_index("s") * NC + lax.axis_index("c")
        pltpu.sync_copy(atab, atab_v)
        pltpu.sync_copy(btab, btab_v)
        sem_i = (sem_i0, sem_i1)
        sem_o = (sem_o0, sem_o1)

        def run(tab_v, idxs, out, n_tasks, nch, ncols, ntc):
            n_max = -(-n_tasks // NW)
            n_pair = -(-n_max // 2)
            total = 2 * n_pair

            def chunk_of(k):
                t = jnp.minimum(wid + k * NW, n_tasks - 1)
                f = t // nch
                n0 = jnp.minimum((t % nch) * CHUNK, ncols - CHUNK)
                return f, n0

            def start_idx(k, s):
                f, n0 = chunk_of(k)
                pltpu.async_copy(idxs.at[pl.ds(f * ncols + n0, CHUNK)],
                                 idx_v.at[s], sem_i[s])

            def wait_idx(s):
                pltpu.make_async_copy(idxs.at[pl.ds(0, CHUNK)],
                                      idx_v.at[s], sem_i[s]).wait()

            def wait_out(s):
                for tr in range(2):
                    pltpu.make_async_copy(rows_v.at[s, tr],
                                          out.at[pl.ds(0, PLANE)],
                                          sem_o[s]).wait()

            start_idx(0, 0)
            start_idx(1, 1)

            def pair_body(kk, carry):
                for s in range(2):
                    k = kk * 2 + s
                    wait_idx(s)

                    @pl.when(k >= 2)
                    def _():
                        wait_out(s)

                    @plsc.parallel_loop(0, CHUNK // LANES, unroll=2)
                    def row_body(j):
                        iv = idx_v[s, pl.ds(j * LANES, LANES)]
                        base = iv * DIM
                        off = (j // 8) * (TILE_R * TILE_C) + (j % 8) * LANES
                        skew = 3
                        vals = []

                        def store(d):
                            rows_v[s, d // TILE_R,
                                   pl.ds(off + (d % TILE_R) * TILE_C,
                                         LANES)] = vals[d]

                        for d in range(DIM):
                            vals.append(plsc.load_gather(tab_v, [base + d]))
                            if d >= skew:
                                store(d - skew)
                        for d in range(DIM - skew, DIM):
                            store(d)

                    f, n0 = chunk_of(k)
                    for tr in range(2):
                        q0 = ((f * 2 + tr) * ntc + n0 // TILE_C) * (TILE_R * TILE_C)
                        pltpu.async_copy(rows_v.at[s, tr],
                                         out.at[pl.ds(q0, PLANE)], sem_o[s])

                    @pl.when(k + 2 < total)
                    def _():
                        start_idx(k + 2, s)
                return carry

            lax.fori_loop(0, n_pair, pair_body, 0)
            for s in range(2):
                wait_out(s)

        run(atab_v, xidx, xout, a_tasks, a_nch, n_atom_cols, a_tc)
        run(btab_v, eidx, eout, e_tasks, e_nch, n_edge_cols, e_tc)

    return gather_kernel


def kernel(x, edge_attr, atom_table, bond_table):
    n_atom, f_atom = x.shape
    n_edge, f_edge = edge_attr.shape
    a_cols = -(-n_atom // TILE_C) * TILE_C
    e_cols = -(-n_edge // TILE_C) * TILE_C

    xt = jnp.pad(x.T.astype(jnp.int32), ((0, 0), (0, a_cols - n_atom)))
    et = jnp.pad(edge_attr.T.astype(jnp.int32), ((0, 0), (0, e_cols - n_edge)))

    gk = _make_gather(a_cols, e_cols, f_atom, f_edge,
                      atom_table.shape[0], bond_table.shape[0])
    xo, eo = gk(atom_table.reshape(-1), xt.reshape(-1),
                bond_table.reshape(-1), et.reshape(-1))

    x_emb = (xo.reshape(f_atom, 2, a_cols // TILE_C, TILE_R, TILE_C)
             .transpose(2, 4, 0, 1, 3)
             .reshape(a_cols, f_atom, DIM)[:n_atom])
    e_emb = (eo.reshape(f_edge, 2, e_cols // TILE_C, TILE_R, TILE_C)
             .transpose(2, 4, 0, 1, 3)
             .reshape(e_cols, f_edge, DIM)[:n_edge])
    return (x_emb, e_emb)
